# baseline (speedup 1.0000x reference)
import functools

import jax
import jax.numpy as jnp
from jax import lax
from jax.experimental import pallas as pl
from jax.experimental.pallas import tpu as pltpu
from jax.experimental.pallas import tpu_sc as plsc

N = 10000
E = 320000
D = 128
H = 8
DH = 16
L = 2

ROWS = 400
GRID = N // ROWS

NW = 32
EB = 80
EPW = E // NW
NBATCH = EPW // EB
RPS = 624
RPS_REM = N - RPS * 16
ND = 632
DPS = 40
NPAD = 10240
BPW = NPAD // NW

_mesh = plsc.VectorSubcoreMesh(core_axis_name="c", subcore_axis_name="s")
_sc_params = pltpu.CompilerParams(needs_layout_passes=False)


def _emb_gather(table, value_pad):
    @functools.partial(
        pl.kernel,
        out_type=jax.ShapeDtypeStruct((NPAD, D), jnp.float32),
        mesh=_mesh,
        scratch_types=[
            pltpu.VMEM((EB,), jnp.int32),
            pltpu.VMEM((EB, D), jnp.float32),
            pltpu.SemaphoreType.DMA,
        ],
        compiler_params=_sc_params,
    )
    def k(tab, val, out, idx_v, rows_v, sem):
        wid = lax.axis_index("c") * 16 + lax.axis_index("s")
        for b in range(BPW // EB):
            base = wid * BPW + b * EB
            pltpu.sync_copy(val.at[pl.ds(base, EB)], idx_v)
            pltpu.async_copy(tab.at[idx_v], rows_v, sem).wait()
            pltpu.sync_copy(rows_v, out.at[pl.ds(base, EB)])

    return k(table, value_pad)


def _edge_sc(q, kv, src, dst, zeros_w):
    @functools.partial(
        pl.kernel,
        out_type=(jax.ShapeDtypeStruct((2, N, D), jnp.float32),
                  jax.ShapeDtypeStruct((2, ND, D), jnp.float32),
                  jax.ShapeDtypeStruct((E * H,), jnp.float32)),
        mesh=_mesh,
        scratch_types=[
            pltpu.VMEM((EB,), jnp.int32),
            pltpu.VMEM((EB,), jnp.int32),
            pltpu.VMEM((EB,), jnp.int32),
            pltpu.VMEM((EB, D), jnp.float32),
            pltpu.VMEM((EB, 2 * D), jnp.float32),
            pltpu.VMEM((EB, D), jnp.float32),
            pltpu.VMEM((EB * H,), jnp.float32),
            pltpu.VMEM_SHARED((N, D), jnp.float32),
            pltpu.VMEM_SHARED((ND, D), jnp.float32),
            pltpu.SemaphoreType.DMA,
            pltpu.SemaphoreType.DMA,
        ],
        compiler_params=_sc_params,
    )
    def k(q_hbm, kv_hbm, src_hbm, dst_hbm, z_hbm, ndw_out, ndd_out, ex_out,
          sidx, didx, didx16, q_v, kv_v, w_v, ex_v,
          acc_w, acc_d, sem1, sem2):
        c = lax.axis_index("c")
        s = lax.axis_index("s")
        wid = c * 16 + s
        pltpu.sync_copy(z_hbm.at[pl.ds(s * RPS, RPS)],
                        acc_w.at[pl.ds(s * RPS, RPS)])

        @pl.when(s == 15)
        def _():
            pltpu.sync_copy(z_hbm.at[pl.ds(16 * RPS, RPS_REM)],
                            acc_w.at[pl.ds(16 * RPS, RPS_REM)])

        dbase = jnp.minimum(s * DPS, ND - DPS)
        pltpu.sync_copy(z_hbm.at[pl.ds(dbase, DPS)],
                        acc_d.at[pl.ds(dbase, DPS)])
        zero16 = jnp.zeros((16,), jnp.float32)
        plsc.subcore_barrier()

        def batch(it, carry):
            ebase = wid * EPW + it * EB
            pltpu.sync_copy(src_hbm.at[pl.ds(ebase, EB)], sidx)
            pltpu.sync_copy(dst_hbm.at[pl.ds(ebase, EB)], didx)
            cp1 = pltpu.async_copy(q_hbm.at[didx], q_v, sem1)
            cp2 = pltpu.async_copy(kv_hbm.at[sidx], kv_v, sem2)
            cp1.wait()
            cp2.wait()

            def group(g, cg):
                e_ids = g * 16 + lax.iota(jnp.int32, 16)
                dstv = plsc.load_gather(didx, [e_ids])
                plsc.store_scatter(didx16, [e_ids],
                                   lax.shift_right_logical(dstv, 4))
                exs = []
                for h in range(H):
                    prods = []
                    for j in range(DH):
                        cj = jnp.full((16,), h * DH + j, jnp.int32)
                        prods.append(plsc.load_gather(q_v, [e_ids, cj])
                                     * plsc.load_gather(kv_v, [e_ids, cj]))
                    while len(prods) > 1:
                        prods = [a + b
                                 for a, b in zip(prods[::2], prods[1::2])]
                    exs.append(jnp.exp(prods[0] * 0.25))
                for h in range(H):
                    plsc.store_scatter(ex_v, [e_ids * H + h], exs[h])
                for h in range(H):
                    for j in range(DH):
                        col = h * DH + j
                        vcol = jnp.full((16,), D + col, jnp.int32)
                        vg = plsc.load_gather(kv_v, [e_ids, vcol])
                        plsc.store_scatter(
                            w_v, [e_ids, jnp.full((16,), col, jnp.int32)],
                            exs[h] * vg)
                return cg

            lax.fori_loop(0, EB // 16, group, 0)
            pltpu.sync_copy(w_v, acc_w.at[didx], add=True)

            def zrow(r, cz):
                for cc in range(D // 16):
                    w_v[r, pl.ds(cc * 16, 16)] = zero16
                return cz

            lax.fori_loop(0, EB, zrow, 0)

            def dgroup(g, cg):
                e_ids = g * 16 + lax.iota(jnp.int32, 16)
                dstv = plsc.load_gather(didx, [e_ids])
                dlow = (dstv & 15) * 8
                for h in range(H):
                    exg = plsc.load_gather(ex_v, [e_ids * H + h])
                    plsc.store_scatter(w_v, [e_ids, dlow + h], exg)
                return cg

            lax.fori_loop(0, EB // 16, dgroup, 0)
            pltpu.sync_copy(w_v, acc_d.at[didx16], add=True)
            pltpu.sync_copy(ex_v, ex_out.at[pl.ds(ebase * H, EB * H)])
            return carry

        lax.fori_loop(0, NBATCH, batch, 0)
        plsc.subcore_barrier()
        pltpu.sync_copy(acc_w.at[pl.ds(s * RPS, RPS)],
                        ndw_out.at[c, pl.ds(s * RPS, RPS)])

        @pl.when(s == 15)
        def _():
            pltpu.sync_copy(acc_w.at[pl.ds(16 * RPS, RPS_REM)],
                            ndw_out.at[c, pl.ds(16 * RPS, RPS_REM)])

        pltpu.sync_copy(acc_d.at[pl.ds(dbase, DPS)],
                        ndd_out.at[c, pl.ds(dbase, DPS)])

    return k(q, kv, src, dst, zeros_w)


def _attn_sc(ex, den, dst):
    @functools.partial(
        pl.kernel,
        out_type=jax.ShapeDtypeStruct((E * H,), jnp.float32),
        mesh=_mesh,
        scratch_types=[
            pltpu.VMEM((N * H,), jnp.float32),
            pltpu.VMEM((EB,), jnp.int32),
            pltpu.VMEM((EB * H,), jnp.float32),
            pltpu.VMEM((EB * H,), jnp.float32),
        ],
        compiler_params=_sc_params,
    )
    def k(ex_hbm, den_hbm, dst_hbm, attn_out, den_v, didx, ex_v, att_v):
        wid = lax.axis_index("c") * 16 + lax.axis_index("s")
        pltpu.sync_copy(den_hbm, den_v)

        def batch(it, carry):
            ebase = wid * EPW + it * EB
            pltpu.sync_copy(dst_hbm.at[pl.ds(ebase, EB)], didx)
            pltpu.sync_copy(ex_hbm.at[pl.ds(ebase * H, EB * H)], ex_v)
            for r in range(EB * H // 16):
                p = r * 16 + lax.iota(jnp.int32, 16)
                e_l = p >> 3
                h_l = p & 7
                dstv = plsc.load_gather(didx, [e_l])
                exv = ex_v[pl.ds(r * 16, 16)]
                denv = plsc.load_gather(den_v, [dstv * H + h_l])
                att_v[pl.ds(r * 16, 16)] = exv / denv
            pltpu.sync_copy(att_v, attn_out.at[pl.ds(ebase * H, EB * H)])
            return carry

        lax.fori_loop(0, NBATCH, batch, 0)

    return k(ex, den, dst)


def _emb_qkv_body(ge, num2, nw, nb, pa, wq, wkv, emb_o, q_o, kv_o):
    h = ge[...] * (num2[...] * nw[...]) + nb[...]
    emb = jnp.where(h > 0, h, pa[0, 0] * h)
    emb_o[...] = emb
    q_o[...] = jnp.dot(emb, wq[...], preferred_element_type=jnp.float32,
                       precision=lax.Precision.HIGHEST)
    kv_o[...] = jnp.dot(emb, wkv[...], preferred_element_type=jnp.float32,
                        precision=lax.Precision.HIGHEST)


def _emb_qkv(ge, num2, nw, nb, pa, wq, wkv):
    row = pl.BlockSpec((ROWS, D), lambda i: (i, 0))
    full = lambda s: pl.BlockSpec(s, lambda i: tuple(0 for _ in s))
    return pl.pallas_call(
        _emb_qkv_body,
        grid=(GRID,),
        in_specs=[row,
                  pl.BlockSpec((ROWS, 1), lambda i: (i, 0)),
                  full((1, D)), full((1, D)), full((1, 1)),
                  full((D, D)), full((D, 2 * D))],
        out_specs=[row, row, pl.BlockSpec((ROWS, 2 * D), lambda i: (i, 0))],
        out_shape=[jax.ShapeDtypeStruct((N, D), jnp.float32),
                   jax.ShapeDtypeStruct((N, D), jnp.float32),
                   jax.ShapeDtypeStruct((N, 2 * D), jnp.float32)],
    )(ge, num2, nw, nb, pa, wq, wkv)


def _agg_ln(feat, num0, num1, den0, den1, lng, lnb):
    densum = den0 + den1 + 1e-9
    numsum = num0 + num1
    cols = []
    for hh in range(H):
        cols.append(numsum[:, hh * DH:(hh + 1) * DH] / densum[:, hh:hh + 1])
    agg = jnp.concatenate(cols, axis=1)
    x = feat + agg
    mu = jnp.mean(x, axis=1, keepdims=True)
    xc = x - mu
    var = jnp.mean(xc * xc, axis=1, keepdims=True)
    ln = xc * lax.rsqrt(var + 1e-5) * lng + lnb
    return ln, densum


def _mid_body(feat, num0, num1, den0, den1, lng, lnb, wq, wkv,
              feat_o, q_o, kv_o):
    ln, _ = _agg_ln(feat[...], num0[0], num1[0], den0[0], den1[0],
                    lng[...], lnb[...])
    feat_o[...] = ln
    q_o[...] = jnp.dot(ln, wq[...], preferred_element_type=jnp.float32,
                       precision=lax.Precision.HIGHEST)
    kv_o[...] = jnp.dot(ln, wkv[...], preferred_element_type=jnp.float32,
                        precision=lax.Precision.HIGHEST)


def _mid(feat, ndw, ndd, lng, lnb, wq, wkv):
    row = pl.BlockSpec((ROWS, D), lambda i: (i, 0))
    w0 = pl.BlockSpec((1, ROWS, D), lambda i: (0, i, 0))
    w1 = pl.BlockSpec((1, ROWS, D), lambda i: (1, i, 0))
    d0 = pl.BlockSpec((1, ROWS, H), lambda i: (0, i, 0))
    d1 = pl.BlockSpec((1, ROWS, H), lambda i: (1, i, 0))
    full = lambda s: pl.BlockSpec(s, lambda i: tuple(0 for _ in s))
    return pl.pallas_call(
        _mid_body,
        grid=(GRID,),
        in_specs=[row, w0, w1, d0, d1, full((1, D)), full((1, D)),
                  full((D, D)), full((D, 2 * D))],
        out_specs=[row, row, pl.BlockSpec((ROWS, 2 * D), lambda i: (i, 0))],
        out_shape=[jax.ShapeDtypeStruct((N, D), jnp.float32),
                   jax.ShapeDtypeStruct((N, D), jnp.float32),
                   jax.ShapeDtypeStruct((N, 2 * D), jnp.float32)],
    )(feat, ndw, ndw, ndd, ndd, lng, lnb, wq, wkv)


def _final_body(feat, num0, num1, den0, den1, lng, lnb, ow, ob, mask,
                feat_o, den_o, out_o):
    ln, densum = _agg_ln(feat[...], num0[0], num1[0], den0[0], den1[0],
                         lng[...], lnb[...])
    feat_o[...] = ln
    den_o[...] = densum
    preds = jnp.sum(ln * ow[...], axis=1, keepdims=True) + ob[0, 0]
    out_o[...] = jnp.where(mask[...], preds, 0.0)


def _final(feat, ndw, ndd, lng, lnb, ow, ob, mask):
    row = pl.BlockSpec((ROWS, D), lambda i: (i, 0))
    row8 = pl.BlockSpec((ROWS, H), lambda i: (i, 0))
    row1 = pl.BlockSpec((ROWS, 1), lambda i: (i, 0))
    w0 = pl.BlockSpec((1, ROWS, D), lambda i: (0, i, 0))
    w1 = pl.BlockSpec((1, ROWS, D), lambda i: (1, i, 0))
    d0 = pl.BlockSpec((1, ROWS, H), lambda i: (0, i, 0))
    d1 = pl.BlockSpec((1, ROWS, H), lambda i: (1, i, 0))
    full = lambda s: pl.BlockSpec(s, lambda i: tuple(0 for _ in s))
    return pl.pallas_call(
        _final_body,
        grid=(GRID,),
        in_specs=[row, w0, w1, d0, d1, full((1, D)), full((1, D)),
                  full((1, D)), full((1, 1)), row1],
        out_specs=[row, row8, row1],
        out_shape=[jax.ShapeDtypeStruct((N, D), jnp.float32),
                   jax.ShapeDtypeStruct((N, H), jnp.float32),
                   jax.ShapeDtypeStruct((N, 1), jnp.float32)],
    )(feat, ndw, ndw, ndd, ndd, lng, lnb, ow, ob, mask)


def kernel(value, number, edge_index, target_mask, targets, emb_table,
           node_w, node_b, prelu_a, Wq, Wk, Wv, ln_g, ln_b, out_W, out_b):
    src = edge_index[0]
    dst = edge_index[1]

    wq = [jnp.transpose(Wq[l].reshape(D, D)) for l in range(L)]
    wkv = [jnp.concatenate([jnp.transpose(Wk[l].reshape(D, D)),
                            jnp.transpose(Wv[l].reshape(D, D))], axis=1)
           for l in range(L)]

    value_pad = jnp.concatenate(
        [value.astype(jnp.int32), jnp.zeros((NPAD - N,), jnp.int32)])
    ge = _emb_gather(emb_table, value_pad)

    zeros_w = jnp.zeros((N, D), jnp.float32)

    emb, q0, kv0 = _emb_qkv(
        ge, number.reshape(N, 1), node_w.reshape(1, D),
        node_b.reshape(1, D), prelu_a.reshape(1, 1), wq[0], wkv[0])

    ndw0, ndd0, _ = _edge_sc(q0, kv0, src, dst, zeros_w)
    feat1, q1, kv1 = _mid(emb, ndw0, ndd0.reshape(2, ND * 16, H),
                          ln_g[0].reshape(1, D), ln_b[0].reshape(1, D),
                          wq[1], wkv[1])

    ndw1, ndd1, ex1 = _edge_sc(q1, kv1, src, dst, zeros_w)
    feat2, dentot, out2 = _final(
        feat1, ndw1, ndd1.reshape(2, ND * 16, H),
        ln_g[1].reshape(1, D), ln_b[1].reshape(1, D),
        out_W.reshape(1, D), out_b.reshape(1, 1),
        target_mask.reshape(N, 1))

    attn = _attn_sc(ex1, dentot.reshape(N * H), dst).reshape(E, H)

    outputs = out2.reshape(N)
    keys = kv1[:, :D].reshape(N, H, DH)
    values = kv1[:, D:].reshape(N, H, DH)
    return (outputs, emb, feat2, keys, values, attn)


# speedup vs baseline: 1.1349x; 1.1349x over previous
import functools

import jax
import jax.numpy as jnp
from jax import lax
from jax.experimental import pallas as pl
from jax.experimental.pallas import tpu as pltpu
from jax.experimental.pallas import tpu_sc as plsc

N = 10000
E = 320000
D = 128
H = 8
DH = 16
L = 2

ROWS = 400
GRID = N // ROWS

NW = 32
EB = 80
EPW = E // NW
NBATCH = EPW // EB
RPS = 624
RPS_REM = N - RPS * 16
ND = 632
DPS = 40
NPAD = 10240
BPW = NPAD // NW

_mesh = plsc.VectorSubcoreMesh(core_axis_name="c", subcore_axis_name="s")
_sc_params = pltpu.CompilerParams(needs_layout_passes=False)


def _emb_gather(table, value_pad):
    @functools.partial(
        pl.kernel,
        out_type=jax.ShapeDtypeStruct((NPAD, D), jnp.float32),
        mesh=_mesh,
        scratch_types=[
            pltpu.VMEM((EB,), jnp.int32),
            pltpu.VMEM((EB, D), jnp.float32),
            pltpu.SemaphoreType.DMA,
        ],
        compiler_params=_sc_params,
    )
    def k(tab, val, out, idx_v, rows_v, sem):
        wid = lax.axis_index("c") * 16 + lax.axis_index("s")
        for b in range(BPW // EB):
            base = wid * BPW + b * EB
            pltpu.sync_copy(val.at[pl.ds(base, EB)], idx_v)
            pltpu.async_copy(tab.at[idx_v], rows_v, sem).wait()
            pltpu.sync_copy(rows_v, out.at[pl.ds(base, EB)])

    return k(table, value_pad)


def _edge_sc(q, kv, src, dst, zeros_w):
    @functools.partial(
        pl.kernel,
        out_type=(jax.ShapeDtypeStruct((2, N, D), jnp.float32),
                  jax.ShapeDtypeStruct((2, ND, D), jnp.float32),
                  jax.ShapeDtypeStruct((E * H,), jnp.float32)),
        mesh=_mesh,
        scratch_types=[
            pltpu.VMEM((EB,), jnp.int32),
            pltpu.VMEM((EB,), jnp.int32),
            pltpu.VMEM((EB,), jnp.int32),
            pltpu.VMEM((EB,), jnp.int32),
            pltpu.VMEM((EB,), jnp.int32),
            pltpu.VMEM((EB,), jnp.int32),
            pltpu.VMEM((EB, D), jnp.float32),
            pltpu.VMEM((EB, 2 * D), jnp.float32),
            pltpu.VMEM((EB, D), jnp.float32),
            pltpu.VMEM((EB * H,), jnp.float32),
            pltpu.VMEM_SHARED((N, D), jnp.float32),
            pltpu.VMEM_SHARED((ND, D), jnp.float32),
            pltpu.SemaphoreType.DMA,
            pltpu.SemaphoreType.DMA,
            pltpu.SemaphoreType.DMA,
        ],
        compiler_params=_sc_params,
    )
    def k(q_hbm, kv_hbm, src_hbm, dst_hbm, z_hbm, ndw_out, ndd_out, ex_out,
          sidx, didx, sidx_p, didx_p, didx_s, didx16, q_v, kv_v, w_v, ex_v,
          acc_w, acc_d, sem1, sem2, sem3):
        c = lax.axis_index("c")
        s = lax.axis_index("s")
        wid = c * 16 + s
        pltpu.sync_copy(z_hbm.at[pl.ds(s * RPS, RPS)],
                        acc_w.at[pl.ds(s * RPS, RPS)])

        @pl.when(s == 15)
        def _():
            pltpu.sync_copy(z_hbm.at[pl.ds(16 * RPS, RPS_REM)],
                            acc_w.at[pl.ds(16 * RPS, RPS_REM)])

        dbase = jnp.minimum(s * DPS, ND - DPS)
        pltpu.sync_copy(z_hbm.at[pl.ds(dbase, DPS)],
                        acc_d.at[pl.ds(dbase, DPS)])
        zero16 = jnp.zeros((16,), jnp.float32)
        plsc.subcore_barrier()

        def compute_groups(didx_c):
            def group(g, cg):
                e_ids = g * 16 + lax.iota(jnp.int32, 16)
                dstv = plsc.load_gather(didx_c, [e_ids])
                plsc.store_scatter(didx16, [e_ids],
                                   lax.shift_right_logical(dstv, 4))
                for h in range(H):
                    acc0 = jnp.zeros((16,), jnp.float32)
                    acc1 = jnp.zeros((16,), jnp.float32)
                    for j in range(0, DH, 2):
                        c0 = jnp.full((16,), h * DH + j, jnp.int32)
                        c1 = jnp.full((16,), h * DH + j + 1, jnp.int32)
                        acc0 += (plsc.load_gather(q_v, [e_ids, c0])
                                 * plsc.load_gather(kv_v, [e_ids, c0]))
                        acc1 += (plsc.load_gather(q_v, [e_ids, c1])
                                 * plsc.load_gather(kv_v, [e_ids, c1]))
                    ex_h = jnp.exp((acc0 + acc1) * 0.25)
                    plsc.store_scatter(ex_v, [e_ids * H + h], ex_h)
                    for j in range(DH):
                        col = h * DH + j
                        vcol = jnp.full((16,), D + col, jnp.int32)
                        vg = plsc.load_gather(kv_v, [e_ids, vcol])
                        plsc.store_scatter(
                            w_v, [e_ids, jnp.full((16,), col, jnp.int32)],
                            ex_h * vg)
                return cg

            lax.fori_loop(0, EB // 16, group, 0)

        def den_rewrite(didx_c):
            def zrow(r, cz):
                for cc in range(D // 16):
                    w_v[r, pl.ds(cc * 16, 16)] = zero16
                return cz

            lax.fori_loop(0, EB, zrow, 0)

            def dgroup(g, cg):
                e_ids = g * 16 + lax.iota(jnp.int32, 16)
                dstv = plsc.load_gather(didx_c, [e_ids])
                dlow = (dstv & 15) * 8
                for h in range(H):
                    exg = plsc.load_gather(ex_v, [e_ids * H + h])
                    plsc.store_scatter(w_v, [e_ids, dlow + h], exg)
                return cg

            lax.fori_loop(0, EB // 16, dgroup, 0)

        def vcopy(src_ref, dst_ref):
            for i in range(EB // 16):
                dst_ref[pl.ds(i * 16, 16)] = src_ref[pl.ds(i * 16, 16)]

        base0 = wid * EPW
        pltpu.sync_copy(src_hbm.at[pl.ds(base0, EB)], sidx)
        pltpu.sync_copy(dst_hbm.at[pl.ds(base0, EB)], didx)
        pltpu.async_copy(q_hbm.at[didx], q_v, sem1)
        pltpu.async_copy(kv_hbm.at[sidx], kv_v, sem2)
        pltpu.async_copy(src_hbm.at[pl.ds(base0 + EB, EB)], sidx_p, sem3)
        pltpu.async_copy(dst_hbm.at[pl.ds(base0 + EB, EB)], didx_p, sem3)

        def batch(it, carry):
            ebase = wid * EPW + it * EB
            pltpu.make_async_copy(q_hbm.at[didx], q_v, sem1).wait()
            pltpu.make_async_copy(kv_hbm.at[sidx], kv_v, sem2).wait()
            compute_groups(didx)
            vcopy(didx, didx_s)

            @pl.when(it < NBATCH - 1)
            def _():
                pltpu.make_async_copy(src_hbm.at[pl.ds(0, EB)],
                                      sidx_p, sem3).wait()
                pltpu.make_async_copy(dst_hbm.at[pl.ds(0, EB)],
                                      didx_p, sem3).wait()
                vcopy(sidx_p, sidx)
                vcopy(didx_p, didx)
                pltpu.async_copy(q_hbm.at[didx], q_v, sem1)
                pltpu.async_copy(kv_hbm.at[sidx], kv_v, sem2)

                @pl.when(it < NBATCH - 2)
                def _():
                    nb = ebase + 2 * EB
                    pltpu.async_copy(src_hbm.at[pl.ds(nb, EB)], sidx_p, sem3)
                    pltpu.async_copy(dst_hbm.at[pl.ds(nb, EB)], didx_p, sem3)

            pltpu.sync_copy(w_v, acc_w.at[didx_s], add=True)
            den_rewrite(didx_s)
            pltpu.sync_copy(w_v, acc_d.at[didx16], add=True)
            pltpu.sync_copy(ex_v, ex_out.at[pl.ds(ebase * H, EB * H)])
            return carry

        lax.fori_loop(0, NBATCH, batch, 0)
        plsc.subcore_barrier()
        pltpu.sync_copy(acc_w.at[pl.ds(s * RPS, RPS)],
                        ndw_out.at[c, pl.ds(s * RPS, RPS)])

        @pl.when(s == 15)
        def _():
            pltpu.sync_copy(acc_w.at[pl.ds(16 * RPS, RPS_REM)],
                            ndw_out.at[c, pl.ds(16 * RPS, RPS_REM)])

        pltpu.sync_copy(acc_d.at[pl.ds(dbase, DPS)],
                        ndd_out.at[c, pl.ds(dbase, DPS)])

    return k(q, kv, src, dst, zeros_w)


def _attn_sc(ex, den, dst):
    @functools.partial(
        pl.kernel,
        out_type=jax.ShapeDtypeStruct((E * H,), jnp.float32),
        mesh=_mesh,
        scratch_types=[
            pltpu.VMEM((N * H,), jnp.float32),
            pltpu.VMEM((EB,), jnp.int32),
            pltpu.VMEM((EB * H,), jnp.float32),
            pltpu.VMEM((EB * H,), jnp.float32),
        ],
        compiler_params=_sc_params,
    )
    def k(ex_hbm, den_hbm, dst_hbm, attn_out, den_v, didx, ex_v, att_v):
        wid = lax.axis_index("c") * 16 + lax.axis_index("s")
        pltpu.sync_copy(den_hbm, den_v)

        def batch(it, carry):
            ebase = wid * EPW + it * EB
            pltpu.sync_copy(dst_hbm.at[pl.ds(ebase, EB)], didx)
            pltpu.sync_copy(ex_hbm.at[pl.ds(ebase * H, EB * H)], ex_v)
            for r in range(EB * H // 16):
                p = r * 16 + lax.iota(jnp.int32, 16)
                e_l = p >> 3
                h_l = p & 7
                dstv = plsc.load_gather(didx, [e_l])
                exv = ex_v[pl.ds(r * 16, 16)]
                denv = plsc.load_gather(den_v, [dstv * H + h_l])
                att_v[pl.ds(r * 16, 16)] = exv / denv
            pltpu.sync_copy(att_v, attn_out.at[pl.ds(ebase * H, EB * H)])
            return carry

        lax.fori_loop(0, NBATCH, batch, 0)

    return k(ex, den, dst)


def _emb_qkv_body(ge, num2, nw, nb, pa, wq, wkv, emb_o, q_o, kv_o):
    h = ge[...] * (num2[...] * nw[...]) + nb[...]
    emb = jnp.where(h > 0, h, pa[0, 0] * h)
    emb_o[...] = emb
    q_o[...] = jnp.dot(emb, wq[...], preferred_element_type=jnp.float32,
                       precision=lax.Precision.HIGHEST)
    kv_o[...] = jnp.dot(emb, wkv[...], preferred_element_type=jnp.float32,
                        precision=lax.Precision.HIGHEST)


def _emb_qkv(ge, num2, nw, nb, pa, wq, wkv):
    row = pl.BlockSpec((ROWS, D), lambda i: (i, 0))
    full = lambda s: pl.BlockSpec(s, lambda i: tuple(0 for _ in s))
    return pl.pallas_call(
        _emb_qkv_body,
        grid=(GRID,),
        in_specs=[row,
                  pl.BlockSpec((ROWS, 1), lambda i: (i, 0)),
                  full((1, D)), full((1, D)), full((1, 1)),
                  full((D, D)), full((D, 2 * D))],
        out_specs=[row, row, pl.BlockSpec((ROWS, 2 * D), lambda i: (i, 0))],
        out_shape=[jax.ShapeDtypeStruct((N, D), jnp.float32),
                   jax.ShapeDtypeStruct((N, D), jnp.float32),
                   jax.ShapeDtypeStruct((N, 2 * D), jnp.float32)],
    )(ge, num2, nw, nb, pa, wq, wkv)


def _agg_ln(feat, num0, num1, den0, den1, lng, lnb):
    densum = den0 + den1 + 1e-9
    numsum = num0 + num1
    cols = []
    for hh in range(H):
        cols.append(numsum[:, hh * DH:(hh + 1) * DH] / densum[:, hh:hh + 1])
    agg = jnp.concatenate(cols, axis=1)
    x = feat + agg
    mu = jnp.mean(x, axis=1, keepdims=True)
    xc = x - mu
    var = jnp.mean(xc * xc, axis=1, keepdims=True)
    ln = xc * lax.rsqrt(var + 1e-5) * lng + lnb
    return ln, densum


def _mid_body(feat, num0, num1, den0, den1, lng, lnb, wq, wkv,
              feat_o, q_o, kv_o):
    ln, _ = _agg_ln(feat[...], num0[0], num1[0], den0[0], den1[0],
                    lng[...], lnb[...])
    feat_o[...] = ln
    q_o[...] = jnp.dot(ln, wq[...], preferred_element_type=jnp.float32,
                       precision=lax.Precision.HIGHEST)
    kv_o[...] = jnp.dot(ln, wkv[...], preferred_element_type=jnp.float32,
                        precision=lax.Precision.HIGHEST)


def _mid(feat, ndw, ndd, lng, lnb, wq, wkv):
    row = pl.BlockSpec((ROWS, D), lambda i: (i, 0))
    w0 = pl.BlockSpec((1, ROWS, D), lambda i: (0, i, 0))
    w1 = pl.BlockSpec((1, ROWS, D), lambda i: (1, i, 0))
    d0 = pl.BlockSpec((1, ROWS, H), lambda i: (0, i, 0))
    d1 = pl.BlockSpec((1, ROWS, H), lambda i: (1, i, 0))
    full = lambda s: pl.BlockSpec(s, lambda i: tuple(0 for _ in s))
    return pl.pallas_call(
        _mid_body,
        grid=(GRID,),
        in_specs=[row, w0, w1, d0, d1, full((1, D)), full((1, D)),
                  full((D, D)), full((D, 2 * D))],
        out_specs=[row, row, pl.BlockSpec((ROWS, 2 * D), lambda i: (i, 0))],
        out_shape=[jax.ShapeDtypeStruct((N, D), jnp.float32),
                   jax.ShapeDtypeStruct((N, D), jnp.float32),
                   jax.ShapeDtypeStruct((N, 2 * D), jnp.float32)],
    )(feat, ndw, ndw, ndd, ndd, lng, lnb, wq, wkv)


def _final_body(feat, num0, num1, den0, den1, lng, lnb, ow, ob, mask,
                feat_o, den_o, out_o):
    ln, densum = _agg_ln(feat[...], num0[0], num1[0], den0[0], den1[0],
                         lng[...], lnb[...])
    feat_o[...] = ln
    den_o[...] = densum
    preds = jnp.sum(ln * ow[...], axis=1, keepdims=True) + ob[0, 0]
    out_o[...] = jnp.where(mask[...], preds, 0.0)


def _final(feat, ndw, ndd, lng, lnb, ow, ob, mask):
    row = pl.BlockSpec((ROWS, D), lambda i: (i, 0))
    row8 = pl.BlockSpec((ROWS, H), lambda i: (i, 0))
    row1 = pl.BlockSpec((ROWS, 1), lambda i: (i, 0))
    w0 = pl.BlockSpec((1, ROWS, D), lambda i: (0, i, 0))
    w1 = pl.BlockSpec((1, ROWS, D), lambda i: (1, i, 0))
    d0 = pl.BlockSpec((1, ROWS, H), lambda i: (0, i, 0))
    d1 = pl.BlockSpec((1, ROWS, H), lambda i: (1, i, 0))
    full = lambda s: pl.BlockSpec(s, lambda i: tuple(0 for _ in s))
    return pl.pallas_call(
        _final_body,
        grid=(GRID,),
        in_specs=[row, w0, w1, d0, d1, full((1, D)), full((1, D)),
                  full((1, D)), full((1, 1)), row1],
        out_specs=[row, row8, row1],
        out_shape=[jax.ShapeDtypeStruct((N, D), jnp.float32),
                   jax.ShapeDtypeStruct((N, H), jnp.float32),
                   jax.ShapeDtypeStruct((N, 1), jnp.float32)],
    )(feat, ndw, ndw, ndd, ndd, lng, lnb, ow, ob, mask)


def kernel(value, number, edge_index, target_mask, targets, emb_table,
           node_w, node_b, prelu_a, Wq, Wk, Wv, ln_g, ln_b, out_W, out_b):
    src = edge_index[0]
    dst = edge_index[1]

    wq = [jnp.transpose(Wq[l].reshape(D, D)) for l in range(L)]
    wkv = [jnp.concatenate([jnp.transpose(Wk[l].reshape(D, D)),
                            jnp.transpose(Wv[l].reshape(D, D))], axis=1)
           for l in range(L)]

    value_pad = jnp.concatenate(
        [value.astype(jnp.int32), jnp.zeros((NPAD - N,), jnp.int32)])
    ge = _emb_gather(emb_table, value_pad)

    zeros_w = jnp.zeros((N, D), jnp.float32)

    emb, q0, kv0 = _emb_qkv(
        ge, number.reshape(N, 1), node_w.reshape(1, D),
        node_b.reshape(1, D), prelu_a.reshape(1, 1), wq[0], wkv[0])

    ndw0, ndd0, _ = _edge_sc(q0, kv0, src, dst, zeros_w)
    feat1, q1, kv1 = _mid(emb, ndw0, ndd0.reshape(2, ND * 16, H),
                          ln_g[0].reshape(1, D), ln_b[0].reshape(1, D),
                          wq[1], wkv[1])

    ndw1, ndd1, ex1 = _edge_sc(q1, kv1, src, dst, zeros_w)
    feat2, dentot, out2 = _final(
        feat1, ndw1, ndd1.reshape(2, ND * 16, H),
        ln_g[1].reshape(1, D), ln_b[1].reshape(1, D),
        out_W.reshape(1, D), out_b.reshape(1, 1),
        target_mask.reshape(N, 1))

    attn = _attn_sc(ex1, dentot.reshape(N * H), dst).reshape(E, H)

    outputs = out2.reshape(N)
    keys = kv1[:, :D].reshape(N, H, DH)
    values = kv1[:, D:].reshape(N, H, DH)
    return (outputs, emb, feat2, keys, values, attn)
